# baseline (speedup 1.0000x reference)
import functools

import jax
import jax.numpy as jnp
from jax import lax
from jax.experimental import pallas as pl
from jax.experimental.pallas import tpu as pltpu
from jax.experimental.pallas import tpu_sc as plsc

B = 16384
N = 200
D = 64
MAXP = 200


RBL = 512
NG = N // 8
NPAD = 256


def _sortable_keys(x):
    u = lax.bitcast_convert_type(x, jnp.int32)
    sgn = lax.shift_right_arithmetic(u, 31)
    return u ^ lax.shift_right_logical(sgn, 1)


def _rank_body(tw_ref, rank_ref, keys_ref, acc_ref):
    keys_ref[...] = _sortable_keys(jnp.transpose(tw_ref[...], (1, 0)))
    acc_ref[...] = jnp.zeros((N, RBL), jnp.int32)

    iot_loc = lax.broadcasted_iota(jnp.int32, (8, RBL), 0)

    def gbody(g, _):
        base = pl.multiple_of(g * 8, 8)
        vg = keys_ref[pl.ds(base, 8), :]
        vkb = [jnp.broadcast_to(vg[s:s + 1, :], (8, RBL)) for s in range(8)]
        for jt in range(NG):
            tj = keys_ref[jt * 8:(jt + 1) * 8, :]
            thr = tj + jnp.where(jt > g, 1, 0).astype(jnp.int32)
            acc = acc_ref[jt * 8:(jt + 1) * 8, :]
            for s in range(8):
                c = vkb[s] < thr
                acc = jnp.where(c, acc + 1, acc)
            acc_ref[jt * 8:(jt + 1) * 8, :] = acc
        accd = acc_ref[pl.ds(base, 8), :]
        for s in range(8):
            m = (vkb[s] == vg) & (iot_loc > s)
            accd = jnp.where(m, accd + 1, accd)
        acc_ref[pl.ds(base, 8), :] = accd
        return 0

    lax.fori_loop(0, NG, gbody, 0)
    rank_ref[...] = lax.concatenate(
        [jnp.transpose(acc_ref[...], (1, 0)),
         jnp.zeros((RBL, NPAD - N), jnp.int32)], 1)


def _ranks_tc(tw):
    return pl.pallas_call(
        _rank_body,
        out_shape=jax.ShapeDtypeStruct((B, NPAD), jnp.int32),
        grid=(B // RBL,),
        in_specs=[pl.BlockSpec((RBL, N), lambda i: (i, 0))],
        out_specs=pl.BlockSpec((RBL, NPAD), lambda i: (i, 0)),
        scratch_shapes=[
            pltpu.VMEM((N, RBL), jnp.int32),
            pltpu.VMEM((N, RBL), jnp.int32),
        ],
    )(tw)


NC = 2
NS = 16
NW = NC * NS
TOTAL = B * N
ROWS_W = B // NW
IB = 32
NBUF = 2


def _sc_gather_body(pe_hbm, idx_hbm, out_hbm, pe_sh, idx_v, rows_v, gsem, wsem):
    wid = lax.axis_index("s") * NC + lax.axis_index("c")
    row0 = wid * ROWS_W
    base = row0 * N

    @pl.when(lax.axis_index("s") == 0)
    def _():
        pltpu.sync_copy(pe_hbm, pe_sh)
    plsc.subcore_barrier()

    def outer(ob, carry):
        pltpu.sync_copy(idx_hbm.at[pl.ds(row0 + ob * IB, IB)], idx_v)

        def inner(t, carry2):
            for buf in range(NBUF):
                r = t * NBUF + buf
                i = ob * IB + r
                off = base + i * N

                @pl.when((ob > 0) | (t >= 1))
                def _():
                    pltpu.make_async_copy(
                        rows_v.at[buf, pl.ds(0, N)],
                        out_hbm.at[pl.ds(off, N)], wsem[buf]).wait()

                c0 = pltpu.async_copy(
                    pe_sh.at[idx_v.at[r, pl.ds(0, 128)]],
                    rows_v.at[buf, pl.ds(0, 128)], gsem[buf])
                c1 = pltpu.async_copy(
                    pe_sh.at[idx_v.at[r, pl.ds(128, 128)]],
                    rows_v.at[buf, pl.ds(128, 128)], gsem[buf])
                c0.wait()
                c1.wait()
                pltpu.async_copy(rows_v.at[buf, pl.ds(0, N)],
                                 out_hbm.at[pl.ds(off, N)], wsem[buf])
            return carry2

        lax.fori_loop(0, IB // NBUF, inner, 0)
        return carry

    lax.fori_loop(0, ROWS_W // IB, outer, 0)

    for buf in range(NBUF):
        i_last = ROWS_W - NBUF + buf
        pltpu.make_async_copy(
            rows_v.at[buf, pl.ds(0, N)],
            out_hbm.at[pl.ds(base + i_last * N, N)], wsem[buf]).wait()


_sc_gather = functools.partial(
    pl.kernel,
    out_type=jax.ShapeDtypeStruct((TOTAL, D), jnp.float32),
    mesh=plsc.VectorSubcoreMesh(
        core_axis_name="c", subcore_axis_name="s", num_cores=NC,
        num_subcores=NS),
    scratch_types=[
        pltpu.VMEM_SHARED((MAXP, D), jnp.float32),
        pltpu.VMEM((IB, NPAD), jnp.int32),
        pltpu.VMEM((NBUF, NPAD, D), jnp.float32),
        [pltpu.SemaphoreType.DMA] * NBUF,
        [pltpu.SemaphoreType.DMA] * NBUF,
    ],
    compiler_params=pltpu.CompilerParams(use_tc_tiling_on_sc=False),
)(_sc_gather_body)


def kernel(tw_start, pe):
    rank = _ranks_tc(tw_start)
    out = _sc_gather(pe, rank)
    return out.reshape(B, N, D)


# speedup vs baseline: 1.1102x; 1.1102x over previous
import functools

import jax
import jax.numpy as jnp
from jax import lax
from jax.experimental import pallas as pl
from jax.experimental.pallas import tpu as pltpu
from jax.experimental.pallas import tpu_sc as plsc

B = 16384
N = 200
D = 64
MAXP = 200


RBL = 512
NG = N // 8
NPAD = 256


def _sortable_keys(x):
    u = lax.bitcast_convert_type(x, jnp.int32)
    sgn = lax.shift_right_arithmetic(u, 31)
    return u ^ lax.shift_right_logical(sgn, 1)


def _rank_body(tw_ref, rank_ref, keys_ref, acc_ref):
    keys_ref[...] = _sortable_keys(jnp.transpose(tw_ref[...], (1, 0)))
    acc_ref[...] = jnp.zeros((N, RBL), jnp.int32)

    iot_loc = lax.broadcasted_iota(jnp.int32, (8, RBL), 0)

    def gbody(g, _):
        base = pl.multiple_of(g * 8, 8)
        vg = keys_ref[pl.ds(base, 8), :]
        vkb = [jnp.broadcast_to(vg[s:s + 1, :], (8, RBL)) for s in range(8)]
        for jt in range(NG):
            tj = keys_ref[jt * 8:(jt + 1) * 8, :]
            thr = tj + jnp.where(jt > g, 1, 0).astype(jnp.int32)
            acc = acc_ref[jt * 8:(jt + 1) * 8, :]
            for s in range(8):
                c = vkb[s] < thr
                acc = jnp.where(c, acc + 1, acc)
            acc_ref[jt * 8:(jt + 1) * 8, :] = acc
        accd = acc_ref[pl.ds(base, 8), :]
        for s in range(8):
            m = (vkb[s] == vg) & (iot_loc > s)
            accd = jnp.where(m, accd + 1, accd)
        acc_ref[pl.ds(base, 8), :] = accd
        return 0

    lax.fori_loop(0, NG, gbody, 0)
    rank_ref[...] = lax.concatenate(
        [jnp.transpose(acc_ref[...], (1, 0)),
         jnp.zeros((RBL, NPAD - N), jnp.int32)], 1)


def _ranks_tc(tw):
    return pl.pallas_call(
        _rank_body,
        out_shape=jax.ShapeDtypeStruct((B, NPAD), jnp.int32),
        grid=(B // RBL,),
        in_specs=[pl.BlockSpec((RBL, N), lambda i: (i, 0))],
        out_specs=pl.BlockSpec((RBL, NPAD), lambda i: (i, 0)),
        scratch_shapes=[
            pltpu.VMEM((N, RBL), jnp.int32),
            pltpu.VMEM((N, RBL), jnp.int32),
        ],
    )(tw)


NC = 2
NS = 16
NW = NC * NS
TOTAL = B * N
ROWS_W = B // NW
IB = 32
SB = 4
NBUF = 2


def _sc_gather_body(pe_hbm, idx_hbm, out_hbm, pe_sh, idx_v, rows_v, gsem, wsem):
    wid = lax.axis_index("s") * NC + lax.axis_index("c")
    row0 = wid * ROWS_W
    base = row0 * N

    @pl.when(lax.axis_index("s") == 0)
    def _():
        pltpu.sync_copy(pe_hbm, pe_sh)
    plsc.subcore_barrier()

    def outer(ob, carry):
        pltpu.sync_copy(idx_hbm.at[pl.ds(row0 + ob * IB, IB)], idx_v)

        def inner(t, carry2):
            for buf in range(NBUF):
                st = t * NBUF + buf
                i = ob * IB + st * SB
                off = base + i * N

                @pl.when((ob > 0) | (t >= 1))
                def _():
                    pltpu.make_async_copy(
                        rows_v.at[buf], out_hbm.at[pl.ds(off, SB * N)],
                        wsem[buf]).wait()

                copies = []
                for q in range(SB):
                    r = st * SB + q
                    copies.append(pltpu.async_copy(
                        pe_sh.at[idx_v.at[r, pl.ds(0, 128)]],
                        rows_v.at[buf, pl.ds(q * N, 128)], gsem[buf]))
                    copies.append(pltpu.async_copy(
                        pe_sh.at[idx_v.at[r, pl.ds(128, N - 128)]],
                        rows_v.at[buf, pl.ds(q * N + 128, N - 128)],
                        gsem[buf]))
                for c in copies:
                    c.wait()
                pltpu.async_copy(rows_v.at[buf],
                                 out_hbm.at[pl.ds(off, SB * N)], wsem[buf])
            return carry2

        lax.fori_loop(0, IB // (SB * NBUF), inner, 0)
        return carry

    lax.fori_loop(0, ROWS_W // IB, outer, 0)

    for buf in range(NBUF):
        i_last = ROWS_W - (NBUF - buf) * SB
        pltpu.make_async_copy(
            rows_v.at[buf],
            out_hbm.at[pl.ds(base + i_last * N, SB * N)], wsem[buf]).wait()


_sc_gather = functools.partial(
    pl.kernel,
    out_type=jax.ShapeDtypeStruct((TOTAL, D), jnp.float32),
    mesh=plsc.VectorSubcoreMesh(
        core_axis_name="c", subcore_axis_name="s", num_cores=NC,
        num_subcores=NS),
    scratch_types=[
        pltpu.VMEM_SHARED((MAXP, D), jnp.float32),
        pltpu.VMEM((IB, NPAD), jnp.int32),
        pltpu.VMEM((NBUF, SB * N, D), jnp.float32),
        [pltpu.SemaphoreType.DMA] * NBUF,
        [pltpu.SemaphoreType.DMA] * NBUF,
    ],
    compiler_params=pltpu.CompilerParams(use_tc_tiling_on_sc=False),
)(_sc_gather_body)


def kernel(tw_start, pe):
    rank = _ranks_tc(tw_start)
    out = _sc_gather(pe, rank)
    return out.reshape(B, N, D)
